# baseline (speedup 1.0000x reference)
import functools

import jax
import jax.numpy as jnp
from jax.experimental import pallas as pl
from jax.experimental.pallas import tpu as pltpu

B, Q, D = 64, 8, 1024
E, F = 8, 2048
TOP_K = 2
T = B * Q
FB = 1024
NF = F // FB
NB = E * NF


def _moe_body(x_ref, gw_ref, w1_ref, w3_ref, w2_ref, out_ref, comb_ref,
              xbf_ref, h_ref):
    k = pl.program_id(0)

    @pl.when(k == 0)
    def _router():
        x = x_ref[...]
        xbf_ref[...] = x.astype(jnp.bfloat16)
        logits = jax.lax.dot_general(
            x, gw_ref[...], (((1,), (1,)), ((), ())),
            preferred_element_type=jnp.float32)
        m = jnp.max(logits, axis=-1, keepdims=True)
        ex = jnp.exp(logits - m)
        p = ex / jnp.sum(ex, axis=-1, keepdims=True)
        m1 = jnp.max(p, axis=-1, keepdims=True)
        neg = jnp.full_like(p, -1.0)
        m2 = jnp.max(jnp.where(p < m1, p, neg), axis=-1, keepdims=True)
        sel = p >= m2
        comb_ref[...] = jnp.where(sel, p, 0.0) / (m1 + m2)

    @pl.when(k < NB)
    def _produce():
        e = k // NF
        xb = xbf_ref[...]
        w1b = w1_ref[0].astype(jnp.bfloat16)
        w3b = w3_ref[0].astype(jnp.bfloat16)
        h1 = jax.lax.dot_general(xb, w1b, (((1,), (1,)), ((), ())),
                                 preferred_element_type=jnp.float32)
        h3 = jax.lax.dot_general(xb, w3b, (((1,), (1,)), ((), ())),
                                 preferred_element_type=jnp.float32)
        lane = jax.lax.broadcasted_iota(jnp.int32, (1, E), 1)
        c_col = jnp.sum(jnp.where(lane == e, comb_ref[...], 0.0),
                        axis=1, keepdims=True)
        sig = 0.5 * jnp.tanh(0.5 * h1) + 0.5
        h = ((h1 * sig) * (h3 * c_col)).astype(jnp.bfloat16)
        h_ref[k % 2] = h

    @pl.when(k > 0)
    def _consume():
        w2b = w2_ref[0].astype(jnp.bfloat16)
        contrib = jax.lax.dot_general(h_ref[(k - 1) % 2], w2b,
                                      (((1,), (1,)), ((), ())),
                                      preferred_element_type=jnp.float32)

        @pl.when(k == 1)
        def _init():
            out_ref[...] = contrib

        @pl.when(k > 1)
        def _add():
            out_ref[...] += contrib


def _pidx(k):
    kp = jnp.minimum(k, NB - 1)
    return kp // NF, kp % NF


def _cidx(k):
    kc = jnp.maximum(k - 1, 0)
    return kc // NF, kc % NF


@functools.partial(jax.jit, static_argnums=())
def _moe(x, gate_w, w1, w3, w2):
    return pl.pallas_call(
        _moe_body,
        grid=(NB + 1,),
        in_specs=[
            pl.BlockSpec((T, D), lambda k: (0, 0)),
            pl.BlockSpec((E, D), lambda k: (0, 0)),
            pl.BlockSpec((1, FB, D), lambda k: (*_pidx(k), 0)),
            pl.BlockSpec((1, FB, D), lambda k: (*_pidx(k), 0)),
            pl.BlockSpec((1, D, FB), lambda k: (_cidx(k)[0], 0, _cidx(k)[1])),
        ],
        out_specs=pl.BlockSpec((T, D), lambda k: (0, 0)),
        out_shape=jax.ShapeDtypeStruct((T, D), jnp.float32),
        scratch_shapes=[
            pltpu.VMEM((T, E), jnp.float32),
            pltpu.VMEM((T, D), jnp.bfloat16),
            pltpu.VMEM((2, T, FB), jnp.bfloat16),
        ],
    )(x, gate_w, w1, w3, w2)


def kernel(hidden_states, gate_w, w1, w3, w2):
    orig_shape = hidden_states.shape
    x = hidden_states.reshape(-1, orig_shape[-1])
    out = _moe(x, gate_w, w1, w3, w2)
    return out.reshape(orig_shape)


# speedup vs baseline: 1.0221x; 1.0221x over previous
import functools

import jax
import jax.numpy as jnp
from jax.experimental import pallas as pl
from jax.experimental.pallas import tpu as pltpu

B, Q, D = 64, 8, 1024
E, F = 8, 2048
TOP_K = 2
T = B * Q
FB = 1024
NF = F // FB


def _moe_body(x_ref, gw_ref, w1_ref, w3_ref, w2_ref, out_ref, comb_ref, xbf_ref):
    e = pl.program_id(0)
    f = pl.program_id(1)
    is_first = (e == 0) & (f == 0)

    @pl.when(is_first)
    def _router():
        x = x_ref[...]
        xbf_ref[...] = x.astype(jnp.bfloat16)
        logits = jax.lax.dot_general(
            x, gw_ref[...], (((1,), (1,)), ((), ())),
            preferred_element_type=jnp.float32)
        m = jnp.max(logits, axis=-1, keepdims=True)
        ex = jnp.exp(logits - m)
        p = ex / jnp.sum(ex, axis=-1, keepdims=True)
        m1 = jnp.max(p, axis=-1, keepdims=True)
        neg = jnp.full_like(p, -1.0)
        m2 = jnp.max(jnp.where(p < m1, p, neg), axis=-1, keepdims=True)
        sel = p >= m2
        comb_ref[...] = jnp.where(sel, p, 0.0) / (m1 + m2)

    xb = xbf_ref[...]
    w1b = w1_ref[0].astype(jnp.bfloat16)
    w3b = w3_ref[0].astype(jnp.bfloat16)
    w2b = w2_ref[0].astype(jnp.bfloat16)
    h1 = jax.lax.dot_general(xb, w1b, (((1,), (1,)), ((), ())),
                             preferred_element_type=jnp.float32)
    h3 = jax.lax.dot_general(xb, w3b, (((1,), (1,)), ((), ())),
                             preferred_element_type=jnp.float32)
    lane = jax.lax.broadcasted_iota(jnp.int32, (1, E), 1)
    c_col = jnp.sum(jnp.where(lane == e, comb_ref[...], 0.0),
                    axis=1, keepdims=True)
    sig = 0.5 * jnp.tanh(0.5 * h1) + 0.5
    h = ((h1 * sig) * (h3 * c_col)).astype(jnp.bfloat16)
    contrib = jax.lax.dot_general(h, w2b,
                                  (((1,), (1,)), ((), ())),
                                  preferred_element_type=jnp.float32)

    @pl.when(is_first)
    def _init():
        out_ref[...] = contrib

    @pl.when(jnp.logical_not(is_first))
    def _add():
        out_ref[...] += contrib


@functools.partial(jax.jit, static_argnums=())
def _moe(x, gate_w, w1, w3, w2):
    return pl.pallas_call(
        _moe_body,
        grid=(E, NF),
        in_specs=[
            pl.BlockSpec((T, D), lambda e, f: (0, 0)),
            pl.BlockSpec((E, D), lambda e, f: (0, 0)),
            pl.BlockSpec((1, FB, D), lambda e, f: (e, f, 0)),
            pl.BlockSpec((1, FB, D), lambda e, f: (e, f, 0)),
            pl.BlockSpec((1, D, FB), lambda e, f: (e, 0, f)),
        ],
        out_specs=pl.BlockSpec((T, D), lambda e, f: (0, 0)),
        out_shape=jax.ShapeDtypeStruct((T, D), jnp.float32),
        scratch_shapes=[
            pltpu.VMEM((T, E), jnp.float32),
            pltpu.VMEM((T, D), jnp.bfloat16),
        ],
    )(x, gate_w, w1, w3, w2)


def kernel(hidden_states, gate_w, w1, w3, w2):
    orig_shape = hidden_states.shape
    x = hidden_states.reshape(-1, orig_shape[-1])
    out = _moe(x, gate_w, w1, w3, w2)
    return out.reshape(orig_shape)
